# Initial kernel scaffold; baseline (speedup 1.0000x reference)
import jax
import jax.numpy as jnp
from jax.experimental import pallas as pl


def kernel(binMapsrc, binMapdst, xx, yy, sxx, syy, cxx, cyy):
    raise NotImplementedError("write your pallas kernel here")


# speedup vs baseline: 268.3820x; 268.3820x over previous
import numpy as np
import jax
import jax.numpy as jnp
from jax.experimental import pallas as pl

_B, _H, _W = 2, 192, 640
_R = 20
_K = 41 * 41
_RP = 2
_EDGE = 0.95
_ALPHA_DW = 0.7
_PIX_MUL = 1.9
_ALPHA_PAD = 1.6
_INF = 1.0e7

_SMOOTH_W = [
    [float(np.exp(-np.sqrt(dx * dx + dy * dy) * _ALPHA_DW)) for dx in range(-_RP, _RP + 1)]
    for dy in range(-_RP, _RP + 1)
]


def _bnmorph_kernel(src_ref, dst_ref, mx_ref, my_ref, ox_ref, oy_ref, cx_ref, cy_ref):
    src = src_ref[0, 0]
    dst = dst_ref[0, 0]
    H, W = src.shape

    key0 = jnp.where(dst > _EDGE, 0.0, _INF)
    padr = jnp.pad(key0, ((_R, _R), (0, 0)), constant_values=_INF)
    col = None
    for dy in range(-_R, _R + 1):
        a = float(1681 * dy * dy + 41 * (dy + _R))
        t = padr[_R + dy:_R + dy + H, :] + a
        col = t if col is None else jnp.minimum(col, t)

    padc = jnp.pad(col, ((0, 0), (_R, _R)), constant_values=_INF)
    key = None
    for dx in range(-_R, _R + 1):
        b = float(1681 * dx * dx + (dx + _R))
        t = padc[:, _R + dx:_R + dx + W] + b
        key = t if key is None else jnp.minimum(key, t)

    found = (key < 2.0e6) & (src > _EDGE)
    foundf = found.astype(jnp.float32)

    q = jnp.floor((key + 0.5) * (1.0 / 1681.0))
    idx = key - q * 1681.0
    dyq = jnp.floor((idx + 0.5) * (1.0 / 41.0))
    dxq = idx - dyq * 41.0
    dispx = jnp.where(found, dxq - float(_R), 0.0)
    dispy = jnp.where(found, dyq - float(_R), 0.0)

    xg = jax.lax.broadcasted_iota(jnp.int32, (H, W), 1).astype(jnp.float32)
    yg = jax.lax.broadcasted_iota(jnp.int32, (H, W), 0).astype(jnp.float32)

    ox_ref[0, 0] = xg * foundf
    oy_ref[0, 0] = yg * foundf
    cx_ref[0, 0] = (xg + dispx) * foundf
    cy_ref[0, 0] = (yg + dispy) * foundf

    pdx = jnp.pad(dispx, _RP)
    pdy = jnp.pad(dispy, _RP)
    pm = jnp.pad(foundf, _RP)
    numx = jnp.zeros((H, W), jnp.float32)
    numy = jnp.zeros((H, W), jnp.float32)
    den = jnp.zeros((H, W), jnp.float32)
    for dy in range(-_RP, _RP + 1):
        for dx in range(-_RP, _RP + 1):
            w = _SMOOTH_W[dy + _RP][dx + _RP]
            numx = numx + w * pdx[_RP + dy:_RP + dy + H, _RP + dx:_RP + dx + W]
            numy = numy + w * pdy[_RP + dy:_RP + dy + H, _RP + dx:_RP + dx + W]
            den = den + w * pm[_RP + dy:_RP + dy + H, _RP + dx:_RP + dx + W]

    mx_ref[0, 0] = xg + numx * _PIX_MUL / (den * 24.0 / 24.0 + _ALPHA_PAD)
    my_ref[0, 0] = yg + numy * _PIX_MUL / (den + _ALPHA_PAD)


def kernel(binMapsrc, binMapdst, xx, yy, sxx, syy, cxx, cyy):
    B, C, H, W = binMapsrc.shape
    out = jax.ShapeDtypeStruct((B, C, H, W), jnp.float32)
    spec = pl.BlockSpec((1, 1, H, W), lambda b: (b, 0, 0, 0))
    return pl.pallas_call(
        _bnmorph_kernel,
        grid=(B,),
        in_specs=[spec, spec],
        out_specs=[spec] * 6,
        out_shape=[out] * 6,
    )(binMapsrc, binMapdst)
